# Initial kernel scaffold; baseline (speedup 1.0000x reference)
import jax
import jax.numpy as jnp
from jax.experimental import pallas as pl


def kernel(x, edge_index, batch_size, seq_len, gcn_W0, gcn_b0, ln_g0, ln_b0, sc_W0, sc_b0, gcn_W1, gcn_b1, ln_g1, ln_b1, sc_W1, sc_b1, gcn_W2, gcn_b2, ln_g2, ln_b2, sc_W2, sc_b2, gcn_W3, gcn_b3, ln_g3, ln_b3, conv_W, conv_b, rln_g, rln_b, h_W1, h_b1, h_W2, h_b2):
    raise NotImplementedError("write your pallas kernel here")


# speedup vs baseline: 9.0399x; 9.0399x over previous
import functools

import jax
import jax.numpy as jnp
from jax import lax
from jax.experimental import pallas as pl
from jax.experimental.pallas import tpu as pltpu
from jax.experimental.pallas import tpu_sc as plsc

N = 10000
E_RAW = 320000
EB = 128
NT = 16
NC = 2
E_PAD = 331776
PER_TILE = E_PAD // NT
PER_WORKER = E_PAD // (NT * NC)
DUMMY = N
NROWS = 10112
RPT = NROWS // NT

_MESH = dict(core_axis_name="c", subcore_axis_name="s")
_SC_PARAMS = pltpu.CompilerParams(use_tc_tiling_on_sc=False)


def _zero_acc(gbuf, acc, s, c2):
    def fill(r, _):
        for j in range(c2 // 16):
            gbuf[r, pl.ds(j * 16, 16)] = jnp.zeros((16,), jnp.float32)
        return 0
    lax.fori_loop(0, EB, fill, 0)
    base = s * RPT
    nfull = RPT // EB
    for k in range(nfull):
        pltpu.sync_copy(gbuf, acc.at[pl.ds(base + k * EB, EB)])
    rem = RPT - nfull * EB
    if rem:
        pltpu.sync_copy(gbuf.at[pl.ds(0, rem)], acc.at[pl.ds(base + nfull * EB, rem)])


def _drain_acc(acc, out, c, s):
    base = s * RPT
    pltpu.sync_copy(acc.at[pl.ds(base, RPT)], out.at[c].at[pl.ds(base, RPT)])


def _deg_body(dst_hbm, out_hbm, gbuf, idx_v, acc):
    c = lax.axis_index("c")
    s = lax.axis_index("s")
    _zero_acc(gbuf, acc, s, 16)

    def fill1(r, _):
        gbuf[r, pl.ds(0, 16)] = jnp.ones((16,), jnp.float32)
        return 0

    plsc.subcore_barrier()
    lax.fori_loop(0, EB, fill1, 0)
    wid = s * NC + c

    def step(b, _):
        base = pl.multiple_of(wid * PER_WORKER + b * EB, EB)
        pltpu.sync_copy(dst_hbm.at[pl.ds(base, EB)], idx_v)
        pltpu.sync_copy(gbuf, acc.at[idx_v], add=True)
        return 0

    lax.fori_loop(0, PER_WORKER // EB, step, 0)
    plsc.subcore_barrier()
    _drain_acc(acc, out_hbm, c, s)


def _agg_body(c2, src_hbm, dst_hbm, z2_hbm, out_hbm, gbuf, sidx_v, gidx_v,
              didx_v, acc):
    c = lax.axis_index("c")
    s = lax.axis_index("s")
    _zero_acc(gbuf, acc, s, c2)
    plsc.subcore_barrier()

    def step(b, _):
        base = pl.multiple_of(s * PER_TILE + b * EB, EB)
        pltpu.sync_copy(src_hbm.at[pl.ds(base, EB)], sidx_v)
        pltpu.sync_copy(dst_hbm.at[pl.ds(base, EB)], didx_v)
        for j in range(EB // 16):
            v = sidx_v[pl.ds(j * 16, 16)]
            gidx_v[pl.ds(j * 16, 16)] = v * 2 + c
        pltpu.sync_copy(z2_hbm.at[gidx_v], gbuf)
        pltpu.sync_copy(gbuf, acc.at[didx_v], add=True)
        return 0

    lax.fori_loop(0, PER_TILE // EB, step, 0)
    plsc.subcore_barrier()
    _drain_acc(acc, out_hbm, c, s)


def _make_deg_kernel():
    return pl.kernel(
        _deg_body,
        out_type=jax.ShapeDtypeStruct((NC, NROWS, 16), jnp.float32),
        mesh=plsc.VectorSubcoreMesh(**_MESH),
        compiler_params=_SC_PARAMS,
        scratch_types=[
            pltpu.VMEM((EB, 16), jnp.float32),
            pltpu.VMEM((EB,), jnp.int32),
            pltpu.VMEM_SHARED((NROWS, 16), jnp.float32),
        ],
    )


def _make_agg_kernel(c2):
    return pl.kernel(
        functools.partial(_agg_body, c2),
        out_type=jax.ShapeDtypeStruct((NC, NROWS, c2), jnp.float32),
        mesh=plsc.VectorSubcoreMesh(**_MESH),
        compiler_params=_SC_PARAMS,
        scratch_types=[
            pltpu.VMEM((EB, c2), jnp.float32),
            pltpu.VMEM((EB,), jnp.int32),
            pltpu.VMEM((EB,), jnp.int32),
            pltpu.VMEM((EB,), jnp.int32),
            pltpu.VMEM_SHARED((NROWS, c2), jnp.float32),
        ],
    )


BN = 1000


def _ln(y, g, b, eps=1e-5):
    m = jnp.mean(y, axis=-1, keepdims=True)
    v = jnp.mean((y - m) * (y - m), axis=-1, keepdims=True)
    return (y - m) * lax.rsqrt(v + eps) * g + b


def _leaky(y):
    return jnp.where(y >= 0, y, 0.01 * y)


def _dinv_of(degp):
    deg = degp[0, :, 0:1] + degp[1, :, 0:1]
    return jnp.where(deg > 0, lax.rsqrt(jnp.maximum(deg, 1e-12)), 0.0)


def _row(shape):
    return pl.BlockSpec(shape, lambda i: (i, 0))


def _full(shape):
    return pl.BlockSpec(shape, lambda i: tuple(0 for _ in shape))


def _agg_spec(c2):
    return pl.BlockSpec((NC, BN, c2), lambda i: (0, i, 0))


def _pre_body(degp, x, W0, z0):
    dinv = _dinv_of(degp[...])
    z0[...] = jnp.dot(x[...], W0[...], preferred_element_type=jnp.float32) * dinv


def _b0_body(agg, degp, x, scW, scb, b0, g0, be0, x1, z1):
    dinv = _dinv_of(degp[...])
    a = agg[...]
    aggcat = jnp.concatenate([a[0], a[1]], axis=-1)
    gcn = dinv * aggcat + b0[...]
    out = _leaky(_ln(gcn, g0[...], be0[...]))
    ident = jnp.dot(x[...], scW[...], preferred_element_type=jnp.float32) + scb[...]
    xn = out + ident
    x1[...] = xn
    z1[...] = dinv * xn


def _bmid_body(agg, degp, xp, W, b, g, be, scW, scb, xn_ref, zn_ref):
    dinv = _dinv_of(degp[...])
    a = agg[...]
    aggs = dinv * jnp.concatenate([a[0], a[1]], axis=-1)
    gcn = jnp.dot(aggs, W[...], preferred_element_type=jnp.float32) + b[...]
    out = _leaky(_ln(gcn, g[...], be[...]))
    ident = jnp.dot(xp[...], scW[...], preferred_element_type=jnp.float32) + scb[...]
    xn = out + ident
    xn_ref[...] = xn
    zn_ref[...] = dinv * xn


def _b3_body(agg, degp, xp, W, b, g, be, x4_ref):
    dinv = _dinv_of(degp[...])
    a = agg[...]
    aggs = dinv * jnp.concatenate([a[0], a[1]], axis=-1)
    gcn = jnp.dot(aggs, W[...], preferred_element_type=jnp.float32) + b[...]
    out = _leaky(_ln(gcn, g[...], be[...]))
    x4_ref[...] = out + xp[...]


def _head_body(x4, Wt, cb, rg, rb, hW1, hb1, hW2, hb2, out_ref):
    xb = x4[0]
    zpad = jnp.zeros((2, 256), jnp.float32)
    xpad = jnp.concatenate([zpad, xb, zpad], axis=0)
    y = jnp.zeros((BN, 128), jnp.float32)
    for k in range(5):
        y = y + jnp.dot(lax.slice(xpad, (k, 0), (k + BN, 256)), Wt[...][k],
                        preferred_element_type=jnp.float32)
    y = _leaky(y + cb[...])
    y = _ln(y, rg[...], rb[...])
    mask = jnp.max(jnp.abs(y), axis=-1, keepdims=True) > 1e-8
    y = y * mask.astype(jnp.float32)
    h = _leaky(jnp.dot(y, hW1[...], preferred_element_type=jnp.float32) + hb1[...])
    out_ref[0] = jnp.dot(h, hW2[...], preferred_element_type=jnp.float32) + hb2[...]


def _tc(body, in_specs, out_specs, out_shape, grid=(10,)):
    return pl.pallas_call(body, grid=grid, in_specs=in_specs,
                          out_specs=out_specs, out_shape=out_shape)


def kernel(x, edge_index, batch_size, seq_len,
           gcn_W0, gcn_b0, ln_g0, ln_b0, sc_W0, sc_b0,
           gcn_W1, gcn_b1, ln_g1, ln_b1, sc_W1, sc_b1,
           gcn_W2, gcn_b2, ln_g2, ln_b2, sc_W2, sc_b2,
           gcn_W3, gcn_b3, ln_g3, ln_b3,
           conv_W, conv_b, rln_g, rln_b, h_W1, h_b1, h_W2, h_b2):
    f32 = jnp.float32
    loop = jnp.arange(N, dtype=jnp.int32)
    npad = E_PAD - E_RAW - N
    src_all = jnp.concatenate([edge_index[0], loop,
                               jnp.zeros((npad,), jnp.int32)])
    dst_all = jnp.concatenate([edge_index[1], loop,
                               jnp.full((npad,), DUMMY, jnp.int32)])

    degp = _make_deg_kernel()(dst_all)

    row1 = lambda n: jnp.reshape(n, (1, -1)).astype(f32)
    b0, g0, be0 = row1(gcn_b0), row1(ln_g0), row1(ln_b0)
    b1, g1, be1 = row1(gcn_b1), row1(ln_g1), row1(ln_b1)
    b2, g2, be2 = row1(gcn_b2), row1(ln_g2), row1(ln_b2)
    b3, g3, be3 = row1(gcn_b3), row1(ln_g3), row1(ln_b3)
    scb0, scb1, scb2 = row1(sc_b0), row1(sc_b1), row1(sc_b2)

    degp_spec = _agg_spec(16)

    z0 = _tc(_pre_body,
             [degp_spec, _row((BN, 128)), _full((128, 64))],
             _row((BN, 64)), jax.ShapeDtypeStruct((N, 64), f32))(degp, x, gcn_W0)

    agg0 = _make_agg_kernel(32)(src_all, dst_all, z0.reshape(2 * N, 32))

    x1, z1 = _tc(_b0_body,
                 [_agg_spec(32), degp_spec, _row((BN, 128)), _full((128, 64)),
                  _full((1, 64)), _full((1, 64)), _full((1, 64)), _full((1, 64))],
                 [_row((BN, 64)), _row((BN, 64))],
                 [jax.ShapeDtypeStruct((N, 64), f32)] * 2)(
                     agg0, degp, x, sc_W0, scb0, b0, g0, be0)

    agg1 = _make_agg_kernel(32)(src_all, dst_all, z1.reshape(2 * N, 32))

    x2, z2 = _tc(_bmid_body,
                 [_agg_spec(32), degp_spec, _row((BN, 64)), _full((64, 128)),
                  _full((1, 128)), _full((1, 128)), _full((1, 128)),
                  _full((64, 128)), _full((1, 128))],
                 [_row((BN, 128)), _row((BN, 128))],
                 [jax.ShapeDtypeStruct((N, 128), f32)] * 2)(
                     agg1, degp, x1, gcn_W1, b1, g1, be1, sc_W1, scb1)

    agg2 = _make_agg_kernel(64)(src_all, dst_all, z2.reshape(2 * N, 64))

    x3, z3 = _tc(_bmid_body,
                 [_agg_spec(64), degp_spec, _row((BN, 128)), _full((128, 256)),
                  _full((1, 256)), _full((1, 256)), _full((1, 256)),
                  _full((128, 256)), _full((1, 256))],
                 [_row((BN, 256)), _row((BN, 256))],
                 [jax.ShapeDtypeStruct((N, 256), f32)] * 2)(
                     agg2, degp, x2, gcn_W2, b2, g2, be2, sc_W2, scb2)

    agg3 = _make_agg_kernel(128)(src_all, dst_all, z3.reshape(2 * N, 128))

    x4 = _tc(_b3_body,
             [_agg_spec(128), degp_spec, _row((BN, 256)), _full((256, 256)),
              _full((1, 256)), _full((1, 256)), _full((1, 256))],
             _row((BN, 256)), jax.ShapeDtypeStruct((N, 256), f32))(
                 agg3, degp, x3, gcn_W3, b3, g3, be3)

    Wt = conv_W.transpose(2, 1, 0)
    coords = _tc(_head_body,
                 [pl.BlockSpec((1, BN, 256), lambda i: (i, 0, 0)),
                  _full((5, 256, 128)), _full((1, 128)), _full((1, 128)),
                  _full((1, 128)), _full((128, 64)), _full((1, 64)),
                  _full((64, 3)), _full((1, 3))],
                 pl.BlockSpec((1, BN, 3), lambda i: (i, 0, 0)),
                 jax.ShapeDtypeStruct((10, BN, 3), f32))(
                     x4.reshape(10, BN, 256), Wt, row1(conv_b), row1(rln_g),
                     row1(rln_b), h_W1, row1(h_b1), h_W2, row1(h_b2))
    return coords


# speedup vs baseline: 15.0224x; 1.6618x over previous
import functools

import jax
import jax.numpy as jnp
from jax import lax
from jax.experimental import pallas as pl
from jax.experimental.pallas import tpu as pltpu
from jax.experimental.pallas import tpu_sc as plsc

N = 10000
E_RAW = 320000
EB = 128
NT = 16
NC = 2
E_PAD = 331776
PER_TILE = E_PAD // NT
PER_WORKER = E_PAD // (NT * NC)
DUMMY = N
NROWS = 10112
RPT = NROWS // NT

_MESH = dict(core_axis_name="c", subcore_axis_name="s")
_SC_PARAMS = pltpu.CompilerParams(use_tc_tiling_on_sc=False)


def _zero_acc(gbuf, acc, s, c2):
    def fill(r, _):
        for j in range(c2 // 16):
            gbuf[r, pl.ds(j * 16, 16)] = jnp.zeros((16,), jnp.float32)
        return 0
    lax.fori_loop(0, EB, fill, 0)
    base = s * RPT
    nfull = RPT // EB
    for k in range(nfull):
        pltpu.sync_copy(gbuf, acc.at[pl.ds(base + k * EB, EB)])
    rem = RPT - nfull * EB
    if rem:
        pltpu.sync_copy(gbuf.at[pl.ds(0, rem)], acc.at[pl.ds(base + nfull * EB, rem)])


def _drain_acc(acc, out2d, s):
    base = s * RPT
    pltpu.sync_copy(acc.at[pl.ds(base, RPT)], out2d.at[pl.ds(base, RPT)])


def _deg_body(dst_hbm, out_hbm, gbuf, idx_v, acc):
    c = lax.axis_index("c")
    s = lax.axis_index("s")
    _zero_acc(gbuf, acc, s, 16)

    def fill1(r, _):
        gbuf[r, pl.ds(0, 16)] = jnp.ones((16,), jnp.float32)
        return 0

    plsc.subcore_barrier()
    lax.fori_loop(0, EB, fill1, 0)
    wid = s * NC + c

    def step(b, _):
        base = pl.multiple_of(wid * PER_WORKER + b * EB, EB)
        pltpu.sync_copy(dst_hbm.at[pl.ds(base, EB)], idx_v)
        pltpu.sync_copy(gbuf, acc.at[idx_v], add=True)
        return 0

    lax.fori_loop(0, PER_WORKER // EB, step, 0)
    plsc.subcore_barrier()
    _drain_acc(acc, out_hbm.at[c], s)


KR = 6
NBLK = PER_TILE // EB


def _agg_body(cw, npass, ed2_hbm, z2_hbm, out_hbm, gbuf, eidx, gidx, didx, acc,
              sem_i, sem_g, sem_s):
    c = lax.axis_index("c")
    s = lax.axis_index("s")
    tbase = s * NBLK
    stride = 2 * npass

    for p in range(npass):
        for k in range(KR):
            pltpu.async_copy(ed2_hbm.at[tbase + k], eidx.at[k], sem_i)

        _zero_acc(gbuf.at[0], acc, s, cw)
        plsc.subcore_barrier()
        sub = 2 * p + c

        @pl.loop(0, NBLK, step=KR)
        def _round(g):
            for k in range(KR):
                pltpu.make_async_copy(ed2_hbm.at[0], eidx.at[k], sem_i).wait()
            for k in range(KR):
                for j in range(EB // 16):
                    v = eidx[k, 0, pl.ds(j * 16, 16)]
                    gidx[k, pl.ds(j * 16, 16)] = v * stride + sub
                    didx[k, pl.ds(j * 16, 16)] = eidx[k, 1, pl.ds(j * 16, 16)]

            @pl.when(g + KR < NBLK)
            def _prefetch():
                for k in range(KR):
                    pltpu.async_copy(ed2_hbm.at[tbase + g + KR + k],
                                     eidx.at[k], sem_i)

            gds = [pltpu.async_copy(z2_hbm.at[gidx.at[k]], gbuf.at[k], sem_g)
                   for k in range(KR)]
            for d in gds:
                d.wait()
            sds = [pltpu.async_copy(gbuf.at[k], acc.at[didx.at[k]], sem_s,
                                    add=True) for k in range(KR)]
            for d in sds:
                d.wait()

        plsc.subcore_barrier()
        _drain_acc(acc, out_hbm.at[c, p], s)


def _make_deg_kernel():
    return pl.kernel(
        _deg_body,
        out_type=jax.ShapeDtypeStruct((NC, NROWS, 16), jnp.float32),
        mesh=plsc.VectorSubcoreMesh(**_MESH),
        compiler_params=_SC_PARAMS,
        scratch_types=[
            pltpu.VMEM((EB, 16), jnp.float32),
            pltpu.VMEM((EB,), jnp.int32),
            pltpu.VMEM_SHARED((NROWS, 16), jnp.float32),
        ],
    )


def _make_agg_kernel(cw, npass):
    return pl.kernel(
        functools.partial(_agg_body, cw, npass),
        out_type=jax.ShapeDtypeStruct((NC, npass, NROWS, cw), jnp.float32),
        mesh=plsc.VectorSubcoreMesh(**_MESH),
        compiler_params=_SC_PARAMS,
        scratch_types=[
            pltpu.VMEM((KR, EB, cw), jnp.float32),
            pltpu.VMEM((KR, 2, EB), jnp.int32),
            pltpu.VMEM((KR, EB), jnp.int32),
            pltpu.VMEM((KR, EB), jnp.int32),
            pltpu.VMEM_SHARED((NROWS, cw), jnp.float32),
            pltpu.SemaphoreType.DMA,
            pltpu.SemaphoreType.DMA,
            pltpu.SemaphoreType.DMA,
        ],
    )


BN = 1000


def _ln(y, g, b, eps=1e-5):
    m = jnp.mean(y, axis=-1, keepdims=True)
    v = jnp.mean((y - m) * (y - m), axis=-1, keepdims=True)
    return (y - m) * lax.rsqrt(v + eps) * g + b


def _leaky(y):
    return jnp.where(y >= 0, y, 0.01 * y)


def _dinv_of(degp):
    deg = degp[0, :, 0:1] + degp[1, :, 0:1]
    return jnp.where(deg > 0, lax.rsqrt(jnp.maximum(deg, 1e-12)), 0.0)


def _row(shape):
    return pl.BlockSpec(shape, lambda i: (i, 0))


def _full(shape):
    return pl.BlockSpec(shape, lambda i: tuple(0 for _ in shape))


def _agg_spec(cw, npass=1):
    return pl.BlockSpec((NC, npass, BN, cw), lambda i: (0, 0, i, 0))


def _cat(a):
    return jnp.concatenate([a[j % 2, j // 2] for j in range(a.shape[0] * a.shape[1])],
                           axis=-1)


def _pre_body(degp, x, W0, z0):
    dinv = _dinv_of(degp[...])
    z0[...] = jnp.dot(x[...], W0[...], preferred_element_type=jnp.float32) * dinv


def _b0_body(agg, degp, x, scW, scb, b0, g0, be0, x1, z1):
    dinv = _dinv_of(degp[...])
    gcn = dinv * _cat(agg[...]) + b0[...]
    out = _leaky(_ln(gcn, g0[...], be0[...]))
    ident = jnp.dot(x[...], scW[...], preferred_element_type=jnp.float32) + scb[...]
    xn = out + ident
    x1[...] = xn
    z1[...] = dinv * xn


def _bmid_body(agg, degp, xp, W, b, g, be, scW, scb, xn_ref, zn_ref):
    dinv = _dinv_of(degp[...])
    aggs = dinv * _cat(agg[...])
    gcn = jnp.dot(aggs, W[...], preferred_element_type=jnp.float32) + b[...]
    out = _leaky(_ln(gcn, g[...], be[...]))
    ident = jnp.dot(xp[...], scW[...], preferred_element_type=jnp.float32) + scb[...]
    xn = out + ident
    xn_ref[...] = xn
    zn_ref[...] = dinv * xn


def _b3_body(agg, degp, xp, W, b, g, be, x4_ref):
    dinv = _dinv_of(degp[...])
    aggs = dinv * _cat(agg[...])
    gcn = jnp.dot(aggs, W[...], preferred_element_type=jnp.float32) + b[...]
    out = _leaky(_ln(gcn, g[...], be[...]))
    x4_ref[...] = out + xp[...]


def _head_body(x4, Wt, cb, rg, rb, hW1, hb1, hW2, hb2, out_ref):
    xb = x4[0]
    zpad = jnp.zeros((2, 256), jnp.float32)
    xpad = jnp.concatenate([zpad, xb, zpad], axis=0)
    y = jnp.zeros((BN, 128), jnp.float32)
    for k in range(5):
        y = y + jnp.dot(lax.slice(xpad, (k, 0), (k + BN, 256)), Wt[...][k],
                        preferred_element_type=jnp.float32)
    y = _leaky(y + cb[...])
    y = _ln(y, rg[...], rb[...])
    mask = jnp.max(jnp.abs(y), axis=-1, keepdims=True) > 1e-8
    y = y * mask.astype(jnp.float32)
    h = _leaky(jnp.dot(y, hW1[...], preferred_element_type=jnp.float32) + hb1[...])
    out_ref[0] = jnp.dot(h, hW2[...], preferred_element_type=jnp.float32) + hb2[...]


def _tc(body, in_specs, out_specs, out_shape, grid=(10,)):
    return pl.pallas_call(body, grid=grid, in_specs=in_specs,
                          out_specs=out_specs, out_shape=out_shape)


def kernel(x, edge_index, batch_size, seq_len,
           gcn_W0, gcn_b0, ln_g0, ln_b0, sc_W0, sc_b0,
           gcn_W1, gcn_b1, ln_g1, ln_b1, sc_W1, sc_b1,
           gcn_W2, gcn_b2, ln_g2, ln_b2, sc_W2, sc_b2,
           gcn_W3, gcn_b3, ln_g3, ln_b3,
           conv_W, conv_b, rln_g, rln_b, h_W1, h_b1, h_W2, h_b2):
    f32 = jnp.float32
    loop = jnp.arange(N, dtype=jnp.int32)
    npad = E_PAD - E_RAW - N
    src_all = jnp.concatenate([edge_index[0], loop,
                               jnp.zeros((npad,), jnp.int32)])
    dst_all = jnp.concatenate([edge_index[1], loop,
                               jnp.full((npad,), DUMMY, jnp.int32)])

    ed2 = jnp.stack([src_all.reshape(-1, EB), dst_all.reshape(-1, EB)], axis=1)

    degp = _make_deg_kernel()(dst_all)

    row1 = lambda n: jnp.reshape(n, (1, -1)).astype(f32)
    b0, g0, be0 = row1(gcn_b0), row1(ln_g0), row1(ln_b0)
    b1, g1, be1 = row1(gcn_b1), row1(ln_g1), row1(ln_b1)
    b2, g2, be2 = row1(gcn_b2), row1(ln_g2), row1(ln_b2)
    b3, g3, be3 = row1(gcn_b3), row1(ln_g3), row1(ln_b3)
    scb0, scb1, scb2 = row1(sc_b0), row1(sc_b1), row1(sc_b2)

    degp_spec = pl.BlockSpec((NC, BN, 16), lambda i: (0, i, 0))

    z0 = _tc(_pre_body,
             [degp_spec, _row((BN, 128)), _full((128, 64))],
             _row((BN, 64)), jax.ShapeDtypeStruct((N, 64), f32))(degp, x, gcn_W0)

    agg0 = _make_agg_kernel(32, 1)(ed2, z0.reshape(2 * N, 32))

    x1, z1 = _tc(_b0_body,
                 [_agg_spec(32, 1), degp_spec, _row((BN, 128)), _full((128, 64)),
                  _full((1, 64)), _full((1, 64)), _full((1, 64)), _full((1, 64))],
                 [_row((BN, 64)), _row((BN, 64))],
                 [jax.ShapeDtypeStruct((N, 64), f32)] * 2)(
                     agg0, degp, x, sc_W0, scb0, b0, g0, be0)

    agg1 = _make_agg_kernel(32, 1)(ed2, z1.reshape(2 * N, 32))

    x2, z2 = _tc(_bmid_body,
                 [_agg_spec(32, 1), degp_spec, _row((BN, 64)), _full((64, 128)),
                  _full((1, 128)), _full((1, 128)), _full((1, 128)),
                  _full((64, 128)), _full((1, 128))],
                 [_row((BN, 128)), _row((BN, 128))],
                 [jax.ShapeDtypeStruct((N, 128), f32)] * 2)(
                     agg1, degp, x1, gcn_W1, b1, g1, be1, sc_W1, scb1)

    agg2 = _make_agg_kernel(32, 2)(ed2, z2.reshape(4 * N, 32))

    x3, z3 = _tc(_bmid_body,
                 [_agg_spec(32, 2), degp_spec, _row((BN, 128)), _full((128, 256)),
                  _full((1, 256)), _full((1, 256)), _full((1, 256)),
                  _full((128, 256)), _full((1, 256))],
                 [_row((BN, 256)), _row((BN, 256))],
                 [jax.ShapeDtypeStruct((N, 256), f32)] * 2)(
                     agg2, degp, x2, gcn_W2, b2, g2, be2, sc_W2, scb2)

    agg3 = _make_agg_kernel(64, 2)(ed2, z3.reshape(4 * N, 64))

    x4 = _tc(_b3_body,
             [_agg_spec(64, 2), degp_spec, _row((BN, 256)), _full((256, 256)),
              _full((1, 256)), _full((1, 256)), _full((1, 256))],
             _row((BN, 256)), jax.ShapeDtypeStruct((N, 256), f32))(
                 agg3, degp, x3, gcn_W3, b3, g3, be3)

    Wt = conv_W.transpose(2, 1, 0)
    coords = _tc(_head_body,
                 [pl.BlockSpec((1, BN, 256), lambda i: (i, 0, 0)),
                  _full((5, 256, 128)), _full((1, 128)), _full((1, 128)),
                  _full((1, 128)), _full((128, 64)), _full((1, 64)),
                  _full((64, 3)), _full((1, 3))],
                 pl.BlockSpec((1, BN, 3), lambda i: (i, 0, 0)),
                 jax.ShapeDtypeStruct((10, BN, 3), f32))(
                     x4.reshape(10, BN, 256), Wt, row1(conv_b), row1(rln_g),
                     row1(rln_b), h_W1, row1(h_b1), h_W2, row1(h_b2))
    return coords


# speedup vs baseline: 15.4611x; 1.0292x over previous
import functools

import jax
import jax.numpy as jnp
from jax import lax
from jax.experimental import pallas as pl
from jax.experimental.pallas import tpu as pltpu
from jax.experimental.pallas import tpu_sc as plsc

N = 10000
E_RAW = 320000
EB = 128
NT = 16
NC = 2
E_PAD = 331776
PER_TILE = E_PAD // NT
PER_WORKER = E_PAD // (NT * NC)
DUMMY = N
NROWS = 10112
RPT = NROWS // NT

_MESH = dict(core_axis_name="c", subcore_axis_name="s")
_SC_PARAMS = pltpu.CompilerParams(use_tc_tiling_on_sc=False)


def _zero_acc(gbuf, acc, s, c2):
    def fill(r, _):
        for j in range(c2 // 16):
            gbuf[r, pl.ds(j * 16, 16)] = jnp.zeros((16,), jnp.float32)
        return 0
    lax.fori_loop(0, EB, fill, 0)
    base = s * RPT
    nfull = RPT // EB
    for k in range(nfull):
        pltpu.sync_copy(gbuf, acc.at[pl.ds(base + k * EB, EB)])
    rem = RPT - nfull * EB
    if rem:
        pltpu.sync_copy(gbuf.at[pl.ds(0, rem)], acc.at[pl.ds(base + nfull * EB, rem)])


def _drain_acc(acc, out2d, s):
    base = s * RPT
    pltpu.sync_copy(acc.at[pl.ds(base, RPT)], out2d.at[pl.ds(base, RPT)])


def _deg_body(dst_hbm, out_hbm, gbuf, idx_v, acc):
    c = lax.axis_index("c")
    s = lax.axis_index("s")
    _zero_acc(gbuf, acc, s, 16)

    def fill1(r, _):
        gbuf[r, pl.ds(0, 16)] = jnp.ones((16,), jnp.float32)
        return 0

    plsc.subcore_barrier()
    lax.fori_loop(0, EB, fill1, 0)
    wid = s * NC + c

    def step(b, _):
        base = pl.multiple_of(wid * PER_WORKER + b * EB, EB)
        pltpu.sync_copy(dst_hbm.at[pl.ds(base, EB)], idx_v)
        pltpu.sync_copy(gbuf, acc.at[idx_v], add=True)
        return 0

    lax.fori_loop(0, PER_WORKER // EB, step, 0)
    plsc.subcore_barrier()
    _drain_acc(acc, out_hbm.at[c], s)


KR = 6
NBLK = PER_TILE // EB


def _agg_body(cw, npass, ed2_hbm, z2_hbm, out_hbm, gbuf, eidx, gidx, didx, acc,
              sem_i, sem_g, sem_s):
    c = lax.axis_index("c")
    s = lax.axis_index("s")
    tbase = s * NBLK
    stride = 2 * npass

    HR = KR // 2
    sem_ih = (sem_i.at[0], sem_i.at[1])
    sem_gh = (sem_g.at[0], sem_g.at[1])
    sem_sh = (sem_s.at[0], sem_s.at[1])

    for p in range(npass):
        for k in range(KR):
            pltpu.async_copy(ed2_hbm.at[tbase + k], eidx.at[k], sem_ih[k // HR])

        _zero_acc(gbuf.at[0], acc, s, cw)
        plsc.subcore_barrier()
        sub = 2 * p + c

        def _stage(h, g):
            ks = range(h * HR, (h + 1) * HR)

            @pl.when(g > 0)
            def _fence_prev_scatters():
                for k in ks:
                    pltpu.make_async_copy(gbuf.at[k], acc.at[didx.at[k]],
                                          sem_sh[h]).wait()
            for k in ks:
                pltpu.make_async_copy(ed2_hbm.at[0], eidx.at[k],
                                      sem_ih[h]).wait()
            for k in ks:
                for j in range(EB // 16):
                    v = eidx[k, 0, pl.ds(j * 16, 16)]
                    gidx[k, pl.ds(j * 16, 16)] = v * stride + sub
                    didx[k, pl.ds(j * 16, 16)] = eidx[k, 1, pl.ds(j * 16, 16)]

            @pl.when(g + KR < NBLK)
            def _prefetch():
                for k in ks:
                    pltpu.async_copy(ed2_hbm.at[tbase + g + KR + k],
                                     eidx.at[k], sem_ih[h])

            return [pltpu.async_copy(z2_hbm.at[gidx.at[k]], gbuf.at[k],
                                     sem_gh[h]) for k in ks]

        @pl.loop(0, NBLK, step=KR)
        def _round(g):
            gds_a = _stage(0, g)
            gds_b = _stage(1, g)
            for d in gds_a:
                d.wait()
            for k in range(0, HR):
                pltpu.async_copy(gbuf.at[k], acc.at[didx.at[k]], sem_sh[0],
                                 add=True)
            for d in gds_b:
                d.wait()
            for k in range(HR, KR):
                pltpu.async_copy(gbuf.at[k], acc.at[didx.at[k]], sem_sh[1],
                                 add=True)

        for k in range(KR):
            pltpu.make_async_copy(gbuf.at[k], acc.at[didx.at[k]],
                                  sem_sh[k // HR]).wait()

        plsc.subcore_barrier()
        _drain_acc(acc, out_hbm.at[c, p], s)


def _make_deg_kernel():
    return pl.kernel(
        _deg_body,
        out_type=jax.ShapeDtypeStruct((NC, NROWS, 16), jnp.float32),
        mesh=plsc.VectorSubcoreMesh(**_MESH),
        compiler_params=_SC_PARAMS,
        scratch_types=[
            pltpu.VMEM((EB, 16), jnp.float32),
            pltpu.VMEM((EB,), jnp.int32),
            pltpu.VMEM_SHARED((NROWS, 16), jnp.float32),
        ],
    )


def _make_agg_kernel(cw, npass):
    return pl.kernel(
        functools.partial(_agg_body, cw, npass),
        out_type=jax.ShapeDtypeStruct((NC, npass, NROWS, cw), jnp.float32),
        mesh=plsc.VectorSubcoreMesh(**_MESH),
        compiler_params=_SC_PARAMS,
        scratch_types=[
            pltpu.VMEM((KR, EB, cw), jnp.float32),
            pltpu.VMEM((KR, 2, EB), jnp.int32),
            pltpu.VMEM((KR, EB), jnp.int32),
            pltpu.VMEM((KR, EB), jnp.int32),
            pltpu.VMEM_SHARED((NROWS, cw), jnp.float32),
            pltpu.SemaphoreType.DMA((2,)),
            pltpu.SemaphoreType.DMA((2,)),
            pltpu.SemaphoreType.DMA((2,)),
        ],
    )


BN = 1000


def _ln(y, g, b, eps=1e-5):
    m = jnp.mean(y, axis=-1, keepdims=True)
    v = jnp.mean((y - m) * (y - m), axis=-1, keepdims=True)
    return (y - m) * lax.rsqrt(v + eps) * g + b


def _leaky(y):
    return jnp.where(y >= 0, y, 0.01 * y)


def _dinv_of(degp):
    deg = degp[0, :, 0:1] + degp[1, :, 0:1]
    return jnp.where(deg > 0, lax.rsqrt(jnp.maximum(deg, 1e-12)), 0.0)


def _row(shape):
    return pl.BlockSpec(shape, lambda i: (i, 0))


def _full(shape):
    return pl.BlockSpec(shape, lambda i: tuple(0 for _ in shape))


def _agg_spec(cw, npass=1):
    return pl.BlockSpec((NC, npass, BN, cw), lambda i: (0, 0, i, 0))


def _cat(a):
    return jnp.concatenate([a[j % 2, j // 2] for j in range(a.shape[0] * a.shape[1])],
                           axis=-1)


def _pre_body(degp, x, W0, z0):
    dinv = _dinv_of(degp[...])
    z0[...] = jnp.dot(x[...], W0[...], preferred_element_type=jnp.float32) * dinv


def _b0_body(agg, degp, x, scW, scb, b0, g0, be0, x1, z1):
    dinv = _dinv_of(degp[...])
    gcn = dinv * _cat(agg[...]) + b0[...]
    out = _leaky(_ln(gcn, g0[...], be0[...]))
    ident = jnp.dot(x[...], scW[...], preferred_element_type=jnp.float32) + scb[...]
    xn = out + ident
    x1[...] = xn
    z1[...] = dinv * xn


def _bmid_body(agg, degp, xp, W, b, g, be, scW, scb, xn_ref, zn_ref):
    dinv = _dinv_of(degp[...])
    aggs = dinv * _cat(agg[...])
    gcn = jnp.dot(aggs, W[...], preferred_element_type=jnp.float32) + b[...]
    out = _leaky(_ln(gcn, g[...], be[...]))
    ident = jnp.dot(xp[...], scW[...], preferred_element_type=jnp.float32) + scb[...]
    xn = out + ident
    xn_ref[...] = xn
    zn_ref[...] = dinv * xn


def _b3_body(agg, degp, xp, W, b, g, be, x4_ref):
    dinv = _dinv_of(degp[...])
    aggs = dinv * _cat(agg[...])
    gcn = jnp.dot(aggs, W[...], preferred_element_type=jnp.float32) + b[...]
    out = _leaky(_ln(gcn, g[...], be[...]))
    x4_ref[...] = out + xp[...]


def _head_body(x4, Wt, cb, rg, rb, hW1, hb1, hW2, hb2, out_ref):
    xb = x4[0]
    zpad = jnp.zeros((2, 256), jnp.float32)
    xpad = jnp.concatenate([zpad, xb, zpad], axis=0)
    y = jnp.zeros((BN, 128), jnp.float32)
    for k in range(5):
        y = y + jnp.dot(lax.slice(xpad, (k, 0), (k + BN, 256)), Wt[...][k],
                        preferred_element_type=jnp.float32)
    y = _leaky(y + cb[...])
    y = _ln(y, rg[...], rb[...])
    mask = jnp.max(jnp.abs(y), axis=-1, keepdims=True) > 1e-8
    y = y * mask.astype(jnp.float32)
    h = _leaky(jnp.dot(y, hW1[...], preferred_element_type=jnp.float32) + hb1[...])
    out_ref[0] = jnp.dot(h, hW2[...], preferred_element_type=jnp.float32) + hb2[...]


def _tc(body, in_specs, out_specs, out_shape, grid=(10,)):
    return pl.pallas_call(body, grid=grid, in_specs=in_specs,
                          out_specs=out_specs, out_shape=out_shape)


def kernel(x, edge_index, batch_size, seq_len,
           gcn_W0, gcn_b0, ln_g0, ln_b0, sc_W0, sc_b0,
           gcn_W1, gcn_b1, ln_g1, ln_b1, sc_W1, sc_b1,
           gcn_W2, gcn_b2, ln_g2, ln_b2, sc_W2, sc_b2,
           gcn_W3, gcn_b3, ln_g3, ln_b3,
           conv_W, conv_b, rln_g, rln_b, h_W1, h_b1, h_W2, h_b2):
    f32 = jnp.float32
    loop = jnp.arange(N, dtype=jnp.int32)
    npad = E_PAD - E_RAW - N
    src_all = jnp.concatenate([edge_index[0], loop,
                               jnp.zeros((npad,), jnp.int32)])
    dst_all = jnp.concatenate([edge_index[1], loop,
                               jnp.full((npad,), DUMMY, jnp.int32)])

    ed2 = jnp.stack([src_all.reshape(-1, EB), dst_all.reshape(-1, EB)], axis=1)

    degp = _make_deg_kernel()(dst_all)

    row1 = lambda n: jnp.reshape(n, (1, -1)).astype(f32)
    b0, g0, be0 = row1(gcn_b0), row1(ln_g0), row1(ln_b0)
    b1, g1, be1 = row1(gcn_b1), row1(ln_g1), row1(ln_b1)
    b2, g2, be2 = row1(gcn_b2), row1(ln_g2), row1(ln_b2)
    b3, g3, be3 = row1(gcn_b3), row1(ln_g3), row1(ln_b3)
    scb0, scb1, scb2 = row1(sc_b0), row1(sc_b1), row1(sc_b2)

    degp_spec = pl.BlockSpec((NC, BN, 16), lambda i: (0, i, 0))

    z0 = _tc(_pre_body,
             [degp_spec, _row((BN, 128)), _full((128, 64))],
             _row((BN, 64)), jax.ShapeDtypeStruct((N, 64), f32))(degp, x, gcn_W0)

    agg0 = _make_agg_kernel(32, 1)(ed2, z0.reshape(2 * N, 32))

    x1, z1 = _tc(_b0_body,
                 [_agg_spec(32, 1), degp_spec, _row((BN, 128)), _full((128, 64)),
                  _full((1, 64)), _full((1, 64)), _full((1, 64)), _full((1, 64))],
                 [_row((BN, 64)), _row((BN, 64))],
                 [jax.ShapeDtypeStruct((N, 64), f32)] * 2)(
                     agg0, degp, x, sc_W0, scb0, b0, g0, be0)

    agg1 = _make_agg_kernel(32, 1)(ed2, z1.reshape(2 * N, 32))

    x2, z2 = _tc(_bmid_body,
                 [_agg_spec(32, 1), degp_spec, _row((BN, 64)), _full((64, 128)),
                  _full((1, 128)), _full((1, 128)), _full((1, 128)),
                  _full((64, 128)), _full((1, 128))],
                 [_row((BN, 128)), _row((BN, 128))],
                 [jax.ShapeDtypeStruct((N, 128), f32)] * 2)(
                     agg1, degp, x1, gcn_W1, b1, g1, be1, sc_W1, scb1)

    agg2 = _make_agg_kernel(32, 2)(ed2, z2.reshape(4 * N, 32))

    x3, z3 = _tc(_bmid_body,
                 [_agg_spec(32, 2), degp_spec, _row((BN, 128)), _full((128, 256)),
                  _full((1, 256)), _full((1, 256)), _full((1, 256)),
                  _full((128, 256)), _full((1, 256))],
                 [_row((BN, 256)), _row((BN, 256))],
                 [jax.ShapeDtypeStruct((N, 256), f32)] * 2)(
                     agg2, degp, x2, gcn_W2, b2, g2, be2, sc_W2, scb2)

    agg3 = _make_agg_kernel(64, 2)(ed2, z3.reshape(4 * N, 64))

    x4 = _tc(_b3_body,
             [_agg_spec(64, 2), degp_spec, _row((BN, 256)), _full((256, 256)),
              _full((1, 256)), _full((1, 256)), _full((1, 256))],
             _row((BN, 256)), jax.ShapeDtypeStruct((N, 256), f32))(
                 agg3, degp, x3, gcn_W3, b3, g3, be3)

    Wt = conv_W.transpose(2, 1, 0)
    coords = _tc(_head_body,
                 [pl.BlockSpec((1, BN, 256), lambda i: (i, 0, 0)),
                  _full((5, 256, 128)), _full((1, 128)), _full((1, 128)),
                  _full((1, 128)), _full((128, 64)), _full((1, 64)),
                  _full((64, 3)), _full((1, 3))],
                 pl.BlockSpec((1, BN, 3), lambda i: (i, 0, 0)),
                 jax.ShapeDtypeStruct((10, BN, 3), f32))(
                     x4.reshape(10, BN, 256), Wt, row1(conv_b), row1(rln_g),
                     row1(rln_b), h_W1, row1(h_b1), h_W2, row1(h_b2))
    return coords


# speedup vs baseline: 15.5733x; 1.0073x over previous
import functools

import jax
import jax.numpy as jnp
from jax import lax
from jax.experimental import pallas as pl
from jax.experimental.pallas import tpu as pltpu
from jax.experimental.pallas import tpu_sc as plsc

N = 10000
E_RAW = 320000
EB = 128
NT = 16
NC = 2
E_PAD = 331776
PER_TILE = E_PAD // NT
PER_WORKER = E_PAD // (NT * NC)
DUMMY = N
NROWS = 10112
RPT = NROWS // NT

_MESH = dict(core_axis_name="c", subcore_axis_name="s")
_SC_PARAMS = pltpu.CompilerParams(use_tc_tiling_on_sc=False)


def _zero_acc(gbuf, acc, s, c2):
    def fill(r, _):
        for j in range(c2 // 16):
            gbuf[r, pl.ds(j * 16, 16)] = jnp.zeros((16,), jnp.float32)
        return 0
    lax.fori_loop(0, EB, fill, 0)
    base = s * RPT
    nfull = RPT // EB
    for k in range(nfull):
        pltpu.sync_copy(gbuf, acc.at[pl.ds(base + k * EB, EB)])
    rem = RPT - nfull * EB
    if rem:
        pltpu.sync_copy(gbuf.at[pl.ds(0, rem)], acc.at[pl.ds(base + nfull * EB, rem)])


def _drain_acc(acc, out2d, s):
    base = s * RPT
    pltpu.sync_copy(acc.at[pl.ds(base, RPT)], out2d.at[pl.ds(base, RPT)])


def _deg_body(dst_hbm, out_hbm, gbuf, idx_v, acc):
    c = lax.axis_index("c")
    s = lax.axis_index("s")
    _zero_acc(gbuf, acc, s, 16)

    def fill1(r, _):
        gbuf[r, pl.ds(0, 16)] = jnp.ones((16,), jnp.float32)
        return 0

    plsc.subcore_barrier()
    lax.fori_loop(0, EB, fill1, 0)
    wid = s * NC + c

    def step(b, _):
        base = pl.multiple_of(wid * PER_WORKER + b * EB, EB)
        pltpu.sync_copy(dst_hbm.at[pl.ds(base, EB)], idx_v)
        pltpu.sync_copy(gbuf, acc.at[idx_v], add=True)
        return 0

    lax.fori_loop(0, PER_WORKER // EB, step, 0)
    plsc.subcore_barrier()
    _drain_acc(acc, out_hbm.at[c], s)


KR = 6
NBLK = PER_TILE // EB


def _agg_body(cw, npass, staged, ed2_hbm, z2_hbm, out_hbm, gbuf, eidx, gidx,
              didx, acc, *rest):
    if staged:
        zstage, sem_i, sem_g, sem_s = rest
    else:
        sem_i, sem_g, sem_s = rest
    c = lax.axis_index("c")
    s = lax.axis_index("s")
    tbase = s * NBLK
    stride = 2 * npass

    HR = KR // 2
    sem_ih = (sem_i.at[0], sem_i.at[1])
    sem_gh = (sem_g.at[0], sem_g.at[1])
    sem_sh = (sem_s.at[0], sem_s.at[1])

    for p in range(npass):
        for k in range(KR):
            pltpu.async_copy(ed2_hbm.at[tbase + k], eidx.at[k], sem_ih[k // HR])

        _zero_acc(gbuf.at[0], acc, s, cw)
        sub = 2 * p + c
        if staged:
            pltpu.sync_copy(z2_hbm.at[sub].at[pl.ds(s * RPT, RPT)],
                            zstage.at[pl.ds(s * RPT, RPT)])
        plsc.subcore_barrier()

        def _stage(h, g):
            ks = range(h * HR, (h + 1) * HR)

            @pl.when(g > 0)
            def _fence_prev_scatters():
                for k in ks:
                    pltpu.make_async_copy(gbuf.at[k], acc.at[didx.at[k]],
                                          sem_sh[h]).wait()
            for k in ks:
                pltpu.make_async_copy(ed2_hbm.at[0], eidx.at[k],
                                      sem_ih[h]).wait()
            for k in ks:
                for j in range(EB // 16):
                    v = eidx[k, 0, pl.ds(j * 16, 16)]
                    gidx[k, pl.ds(j * 16, 16)] = (v if staged
                                                  else v * stride + sub)
                    didx[k, pl.ds(j * 16, 16)] = eidx[k, 1, pl.ds(j * 16, 16)]

            @pl.when(g + KR < NBLK)
            def _prefetch():
                for k in ks:
                    pltpu.async_copy(ed2_hbm.at[tbase + g + KR + k],
                                     eidx.at[k], sem_ih[h])

            ztab = zstage if staged else z2_hbm
            return [pltpu.async_copy(ztab.at[gidx.at[k]], gbuf.at[k],
                                     sem_gh[h]) for k in ks]

        @pl.loop(0, NBLK, step=KR)
        def _round(g):
            gds_a = _stage(0, g)
            gds_b = _stage(1, g)
            for d in gds_a:
                d.wait()
            for k in range(0, HR):
                pltpu.async_copy(gbuf.at[k], acc.at[didx.at[k]], sem_sh[0],
                                 add=True)
            for d in gds_b:
                d.wait()
            for k in range(HR, KR):
                pltpu.async_copy(gbuf.at[k], acc.at[didx.at[k]], sem_sh[1],
                                 add=True)

        for k in range(KR):
            pltpu.make_async_copy(gbuf.at[k], acc.at[didx.at[k]],
                                  sem_sh[k // HR]).wait()

        plsc.subcore_barrier()
        _drain_acc(acc, out_hbm.at[c, p], s)


def _make_deg_kernel():
    return pl.kernel(
        _deg_body,
        out_type=jax.ShapeDtypeStruct((NC, NROWS, 16), jnp.float32),
        mesh=plsc.VectorSubcoreMesh(**_MESH),
        compiler_params=_SC_PARAMS,
        scratch_types=[
            pltpu.VMEM((EB, 16), jnp.float32),
            pltpu.VMEM((EB,), jnp.int32),
            pltpu.VMEM_SHARED((NROWS, 16), jnp.float32),
        ],
    )


def _make_agg_kernel(cw, npass, staged=False):
    scratch = [
        pltpu.VMEM((KR, EB, cw), jnp.float32),
        pltpu.VMEM((KR, 2, EB), jnp.int32),
        pltpu.VMEM((KR, EB), jnp.int32),
        pltpu.VMEM((KR, EB), jnp.int32),
        pltpu.VMEM_SHARED((NROWS, cw), jnp.float32),
    ]
    if staged:
        scratch.append(pltpu.VMEM_SHARED((NROWS, cw), jnp.float32))
    scratch += [pltpu.SemaphoreType.DMA((2,))] * 3
    return pl.kernel(
        functools.partial(_agg_body, cw, npass, staged),
        out_type=jax.ShapeDtypeStruct((NC, npass, NROWS, cw), jnp.float32),
        mesh=plsc.VectorSubcoreMesh(**_MESH),
        compiler_params=_SC_PARAMS,
        scratch_types=scratch,
    )


BN = 1000


def _ln(y, g, b, eps=1e-5):
    m = jnp.mean(y, axis=-1, keepdims=True)
    v = jnp.mean((y - m) * (y - m), axis=-1, keepdims=True)
    return (y - m) * lax.rsqrt(v + eps) * g + b


def _leaky(y):
    return jnp.where(y >= 0, y, 0.01 * y)


def _dinv_of(degp):
    deg = degp[0, :, 0:1] + degp[1, :, 0:1]
    return jnp.where(deg > 0, lax.rsqrt(jnp.maximum(deg, 1e-12)), 0.0)


def _row(shape):
    return pl.BlockSpec(shape, lambda i: (i, 0))


def _full(shape):
    return pl.BlockSpec(shape, lambda i: tuple(0 for _ in shape))


def _agg_spec(cw, npass=1):
    return pl.BlockSpec((NC, npass, BN, cw), lambda i: (0, 0, i, 0))


def _cat(a):
    return jnp.concatenate([a[j % 2, j // 2] for j in range(a.shape[0] * a.shape[1])],
                           axis=-1)


def _pre_body(degp, x, W0, z0):
    dinv = _dinv_of(degp[...])
    z0[...] = jnp.dot(x[...], W0[...], preferred_element_type=jnp.float32) * dinv


def _b0_body(agg, degp, x, scW, scb, b0, g0, be0, x1, z1):
    dinv = _dinv_of(degp[...])
    gcn = dinv * _cat(agg[...]) + b0[...]
    out = _leaky(_ln(gcn, g0[...], be0[...]))
    ident = jnp.dot(x[...], scW[...], preferred_element_type=jnp.float32) + scb[...]
    xn = out + ident
    x1[...] = xn
    z1[...] = dinv * xn


def _bmid_body(agg, degp, xp, W, b, g, be, scW, scb, xn_ref, zn_ref):
    dinv = _dinv_of(degp[...])
    aggs = dinv * _cat(agg[...])
    gcn = jnp.dot(aggs, W[...], preferred_element_type=jnp.float32) + b[...]
    out = _leaky(_ln(gcn, g[...], be[...]))
    ident = jnp.dot(xp[...], scW[...], preferred_element_type=jnp.float32) + scb[...]
    xn = out + ident
    xn_ref[...] = xn
    zn_ref[...] = dinv * xn


def _b3_body(agg, degp, xp, W, b, g, be, x4_ref):
    dinv = _dinv_of(degp[...])
    aggs = dinv * _cat(agg[...])
    gcn = jnp.dot(aggs, W[...], preferred_element_type=jnp.float32) + b[...]
    out = _leaky(_ln(gcn, g[...], be[...]))
    x4_ref[...] = out + xp[...]


def _head_body(x4, Wt, cb, rg, rb, hW1, hb1, hW2, hb2, out_ref):
    xb = x4[0]
    zpad = jnp.zeros((2, 256), jnp.float32)
    xpad = jnp.concatenate([zpad, xb, zpad], axis=0)
    y = jnp.zeros((BN, 128), jnp.float32)
    for k in range(5):
        y = y + jnp.dot(lax.slice(xpad, (k, 0), (k + BN, 256)), Wt[...][k],
                        preferred_element_type=jnp.float32)
    y = _leaky(y + cb[...])
    y = _ln(y, rg[...], rb[...])
    mask = jnp.max(jnp.abs(y), axis=-1, keepdims=True) > 1e-8
    y = y * mask.astype(jnp.float32)
    h = _leaky(jnp.dot(y, hW1[...], preferred_element_type=jnp.float32) + hb1[...])
    out_ref[0] = jnp.dot(h, hW2[...], preferred_element_type=jnp.float32) + hb2[...]


def _tc(body, in_specs, out_specs, out_shape, grid=(10,)):
    return pl.pallas_call(body, grid=grid, in_specs=in_specs,
                          out_specs=out_specs, out_shape=out_shape)


def kernel(x, edge_index, batch_size, seq_len,
           gcn_W0, gcn_b0, ln_g0, ln_b0, sc_W0, sc_b0,
           gcn_W1, gcn_b1, ln_g1, ln_b1, sc_W1, sc_b1,
           gcn_W2, gcn_b2, ln_g2, ln_b2, sc_W2, sc_b2,
           gcn_W3, gcn_b3, ln_g3, ln_b3,
           conv_W, conv_b, rln_g, rln_b, h_W1, h_b1, h_W2, h_b2):
    f32 = jnp.float32
    loop = jnp.arange(N, dtype=jnp.int32)
    npad = E_PAD - E_RAW - N
    src_all = jnp.concatenate([edge_index[0], loop,
                               jnp.zeros((npad,), jnp.int32)])
    dst_all = jnp.concatenate([edge_index[1], loop,
                               jnp.full((npad,), DUMMY, jnp.int32)])

    ed2 = jnp.stack([src_all.reshape(-1, EB), dst_all.reshape(-1, EB)], axis=1)

    degp = _make_deg_kernel()(dst_all)

    row1 = lambda n: jnp.reshape(n, (1, -1)).astype(f32)
    b0, g0, be0 = row1(gcn_b0), row1(ln_g0), row1(ln_b0)
    b1, g1, be1 = row1(gcn_b1), row1(ln_g1), row1(ln_b1)
    b2, g2, be2 = row1(gcn_b2), row1(ln_g2), row1(ln_b2)
    b3, g3, be3 = row1(gcn_b3), row1(ln_g3), row1(ln_b3)
    scb0, scb1, scb2 = row1(sc_b0), row1(sc_b1), row1(sc_b2)

    degp_spec = pl.BlockSpec((NC, BN, 16), lambda i: (0, i, 0))

    z0 = _tc(_pre_body,
             [degp_spec, _row((BN, 128)), _full((128, 64))],
             _row((BN, 64)), jax.ShapeDtypeStruct((N, 64), f32))(degp, x, gcn_W0)

    agg0 = _make_agg_kernel(32, 1)(ed2, z0.reshape(2 * N, 32))

    x1, z1 = _tc(_b0_body,
                 [_agg_spec(32, 1), degp_spec, _row((BN, 128)), _full((128, 64)),
                  _full((1, 64)), _full((1, 64)), _full((1, 64)), _full((1, 64))],
                 [_row((BN, 64)), _row((BN, 64))],
                 [jax.ShapeDtypeStruct((N, 64), f32)] * 2)(
                     agg0, degp, x, sc_W0, scb0, b0, g0, be0)

    agg1 = _make_agg_kernel(32, 1)(ed2, z1.reshape(2 * N, 32))

    x2, z2 = _tc(_bmid_body,
                 [_agg_spec(32, 1), degp_spec, _row((BN, 64)), _full((64, 128)),
                  _full((1, 128)), _full((1, 128)), _full((1, 128)),
                  _full((64, 128)), _full((1, 128))],
                 [_row((BN, 128)), _row((BN, 128))],
                 [jax.ShapeDtypeStruct((N, 128), f32)] * 2)(
                     agg1, degp, x1, gcn_W1, b1, g1, be1, sc_W1, scb1)

    agg2 = _make_agg_kernel(32, 2)(ed2, z2.reshape(4 * N, 32))

    x3, z3 = _tc(_bmid_body,
                 [_agg_spec(32, 2), degp_spec, _row((BN, 128)), _full((128, 256)),
                  _full((1, 256)), _full((1, 256)), _full((1, 256)),
                  _full((128, 256)), _full((1, 256))],
                 [_row((BN, 256)), _row((BN, 256))],
                 [jax.ShapeDtypeStruct((N, 256), f32)] * 2)(
                     agg2, degp, x2, gcn_W2, b2, g2, be2, sc_W2, scb2)

    z3p = jnp.pad(z3.reshape(N, 8, 32).transpose(1, 0, 2),
                  ((0, 0), (0, NROWS - N), (0, 0)))
    agg3 = _make_agg_kernel(32, 4, staged=True)(ed2, z3p)

    x4 = _tc(_b3_body,
             [_agg_spec(32, 4), degp_spec, _row((BN, 256)), _full((256, 256)),
              _full((1, 256)), _full((1, 256)), _full((1, 256))],
             _row((BN, 256)), jax.ShapeDtypeStruct((N, 256), f32))(
                 agg3, degp, x3, gcn_W3, b3, g3, be3)

    Wt = conv_W.transpose(2, 1, 0)
    coords = _tc(_head_body,
                 [pl.BlockSpec((1, BN, 256), lambda i: (i, 0, 0)),
                  _full((5, 256, 128)), _full((1, 128)), _full((1, 128)),
                  _full((1, 128)), _full((128, 64)), _full((1, 64)),
                  _full((64, 3)), _full((1, 3))],
                 pl.BlockSpec((1, BN, 3), lambda i: (i, 0, 0)),
                 jax.ShapeDtypeStruct((10, BN, 3), f32))(
                     x4.reshape(10, BN, 256), Wt, row1(conv_b), row1(rln_g),
                     row1(rln_b), h_W1, row1(h_b1), h_W2, row1(h_b2))
    return coords


# speedup vs baseline: 16.3244x; 1.0482x over previous
import functools

import jax
import jax.numpy as jnp
from jax import lax
from jax.experimental import pallas as pl
from jax.experimental.pallas import tpu as pltpu
from jax.experimental.pallas import tpu_sc as plsc

N = 10000
E_RAW = 320000
EB = 128
NT = 16
NC = 2
E_PAD = 331776
PER_TILE = E_PAD // NT
PER_WORKER = E_PAD // (NT * NC)
DUMMY = N
NROWS = 10112
RPT = NROWS // NT

_MESH = dict(core_axis_name="c", subcore_axis_name="s")
_SC_PARAMS = pltpu.CompilerParams(use_tc_tiling_on_sc=False)


def _zero_acc(gbuf, acc, s, c2):
    def fill(r, _):
        for j in range(c2 // 16):
            gbuf[r, pl.ds(j * 16, 16)] = jnp.zeros((16,), jnp.float32)
        return 0
    lax.fori_loop(0, EB, fill, 0)
    base = s * RPT
    nfull = RPT // EB
    for k in range(nfull):
        pltpu.sync_copy(gbuf, acc.at[pl.ds(base + k * EB, EB)])
    rem = RPT - nfull * EB
    if rem:
        pltpu.sync_copy(gbuf.at[pl.ds(0, rem)], acc.at[pl.ds(base + nfull * EB, rem)])


def _drain_acc(acc, out2d, s):
    base = s * RPT
    pltpu.sync_copy(acc.at[pl.ds(base, RPT)], out2d.at[pl.ds(base, RPT)])


def _deg_body(dst_hbm, out_hbm, gbuf, idx_v, acc):
    c = lax.axis_index("c")
    s = lax.axis_index("s")
    _zero_acc(gbuf, acc, s, 16)

    def fill1(r, _):
        gbuf[r, pl.ds(0, 16)] = jnp.ones((16,), jnp.float32)
        return 0

    plsc.subcore_barrier()
    lax.fori_loop(0, EB, fill1, 0)
    wid = s * NC + c

    def step(b, _):
        base = pl.multiple_of(wid * PER_WORKER + b * EB, EB)
        pltpu.sync_copy(dst_hbm.at[pl.ds(base, EB)], idx_v)
        pltpu.sync_copy(gbuf, acc.at[idx_v], add=True)
        return 0

    lax.fori_loop(0, PER_WORKER // EB, step, 0)
    plsc.subcore_barrier()
    _drain_acc(acc, out_hbm.at[c], s)


KR = 6
NBLK = PER_TILE // EB


def _agg_body(cw, npass, staged, ed2_hbm, z2_hbm, out_hbm, gbuf, eidx, gidx,
              didx, acc, *rest):
    if staged:
        zstage, sem_i, sem_g, sem_s = rest
    else:
        sem_i, sem_g, sem_s = rest
    c = lax.axis_index("c")
    s = lax.axis_index("s")
    tbase = s * NBLK
    stride = 2 * npass

    HR = KR // 2
    sem_ih = (sem_i.at[0], sem_i.at[1])
    sem_gh = (sem_g.at[0], sem_g.at[1])
    sem_sh = (sem_s.at[0], sem_s.at[1])

    for p in range(npass):
        for k in range(KR):
            pltpu.async_copy(ed2_hbm.at[tbase + k], eidx.at[k], sem_ih[k // HR])

        _zero_acc(gbuf.at[0], acc, s, cw)
        sub = 2 * p + c
        if staged:
            lanes = lax.iota(jnp.int32, 16)
            cap = (N - 1) * stride + sub
            for m, ln in enumerate([EB] * (RPT // EB) + [RPT % EB]):
                if ln == 0:
                    continue
                for j in range((ln + 15) // 16):
                    i0 = s * RPT + m * EB + j * 16
                    idx16 = jnp.minimum((i0 + lanes) * stride + sub, cap)
                    gidx[0, pl.ds(j * 16, 16)] = idx16
                pltpu.sync_copy(z2_hbm.at[gidx.at[0, pl.ds(0, ln)]],
                                gbuf.at[0].at[pl.ds(0, ln)])
                pltpu.sync_copy(gbuf.at[0].at[pl.ds(0, ln)],
                                zstage.at[pl.ds(s * RPT + m * EB, ln)])
        plsc.subcore_barrier()

        def _stage(h, g):
            ks = range(h * HR, (h + 1) * HR)

            @pl.when(g > 0)
            def _fence_prev_scatters():
                for k in ks:
                    pltpu.make_async_copy(gbuf.at[k], acc.at[didx.at[k]],
                                          sem_sh[h]).wait()
            for k in ks:
                pltpu.make_async_copy(ed2_hbm.at[0], eidx.at[k],
                                      sem_ih[h]).wait()
            for k in ks:
                for j in range(EB // 16):
                    v = eidx[k, 0, pl.ds(j * 16, 16)]
                    gidx[k, pl.ds(j * 16, 16)] = (v if staged
                                                  else v * stride + sub)
                    didx[k, pl.ds(j * 16, 16)] = eidx[k, 1, pl.ds(j * 16, 16)]

            @pl.when(g + KR < NBLK)
            def _prefetch():
                for k in ks:
                    pltpu.async_copy(ed2_hbm.at[tbase + g + KR + k],
                                     eidx.at[k], sem_ih[h])

            ztab = zstage if staged else z2_hbm
            return [pltpu.async_copy(ztab.at[gidx.at[k]], gbuf.at[k],
                                     sem_gh[h]) for k in ks]

        @pl.loop(0, NBLK, step=KR)
        def _round(g):
            gds_a = _stage(0, g)
            gds_b = _stage(1, g)
            for d in gds_a:
                d.wait()
            for k in range(0, HR):
                pltpu.async_copy(gbuf.at[k], acc.at[didx.at[k]], sem_sh[0],
                                 add=True)
            for d in gds_b:
                d.wait()
            for k in range(HR, KR):
                pltpu.async_copy(gbuf.at[k], acc.at[didx.at[k]], sem_sh[1],
                                 add=True)

        for k in range(KR):
            pltpu.make_async_copy(gbuf.at[k], acc.at[didx.at[k]],
                                  sem_sh[k // HR]).wait()

        plsc.subcore_barrier()
        _drain_acc(acc, out_hbm.at[c, p], s)


def _make_deg_kernel():
    return pl.kernel(
        _deg_body,
        out_type=jax.ShapeDtypeStruct((NC, NROWS, 16), jnp.float32),
        mesh=plsc.VectorSubcoreMesh(**_MESH),
        compiler_params=_SC_PARAMS,
        scratch_types=[
            pltpu.VMEM((EB, 16), jnp.float32),
            pltpu.VMEM((EB,), jnp.int32),
            pltpu.VMEM_SHARED((NROWS, 16), jnp.float32),
        ],
    )


def _make_agg_kernel(cw, npass, staged=False):
    scratch = [
        pltpu.VMEM((KR, EB, cw), jnp.float32),
        pltpu.VMEM((KR, 2, EB), jnp.int32),
        pltpu.VMEM((KR, EB), jnp.int32),
        pltpu.VMEM((KR, EB), jnp.int32),
        pltpu.VMEM_SHARED((NROWS, cw), jnp.float32),
    ]
    if staged:
        scratch.append(pltpu.VMEM_SHARED((NROWS, cw), jnp.float32))
    scratch += [pltpu.SemaphoreType.DMA((2,))] * 3
    return pl.kernel(
        functools.partial(_agg_body, cw, npass, staged),
        out_type=jax.ShapeDtypeStruct((NC, npass, NROWS, cw), jnp.float32),
        mesh=plsc.VectorSubcoreMesh(**_MESH),
        compiler_params=_SC_PARAMS,
        scratch_types=scratch,
    )


BN = 1000


def _ln(y, g, b, eps=1e-5):
    m = jnp.mean(y, axis=-1, keepdims=True)
    v = jnp.mean((y - m) * (y - m), axis=-1, keepdims=True)
    return (y - m) * lax.rsqrt(v + eps) * g + b


def _leaky(y):
    return jnp.where(y >= 0, y, 0.01 * y)


def _dinv_of(degp):
    deg = degp[0, :, 0:1] + degp[1, :, 0:1]
    return jnp.where(deg > 0, lax.rsqrt(jnp.maximum(deg, 1e-12)), 0.0)


def _row(shape):
    return pl.BlockSpec(shape, lambda i: (i, 0))


def _full(shape):
    return pl.BlockSpec(shape, lambda i: tuple(0 for _ in shape))


def _agg_spec(cw, npass=1):
    return pl.BlockSpec((NC, npass, BN, cw), lambda i: (0, 0, i, 0))


def _cat(a):
    return jnp.concatenate([a[j % 2, j // 2] for j in range(a.shape[0] * a.shape[1])],
                           axis=-1)


def _mm_body(x, W0, xw0):
    xw0[...] = jnp.dot(x[...], W0[...], preferred_element_type=jnp.float32)


def _scale_body(degp, xw0, z0):
    z0[...] = xw0[...] * _dinv_of(degp[...])


def _b0_body(agg, degp, x, scW, scb, b0, g0, be0, x1, z1):
    dinv = _dinv_of(degp[...])
    gcn = dinv * _cat(agg[...]) + b0[...]
    out = _leaky(_ln(gcn, g0[...], be0[...]))
    ident = jnp.dot(x[...], scW[...], preferred_element_type=jnp.float32) + scb[...]
    xn = out + ident
    x1[...] = xn
    z1[...] = dinv * xn


def _bmid_body(agg, degp, xp, W, b, g, be, scW, scb, xn_ref, zn_ref):
    dinv = _dinv_of(degp[...])
    aggs = dinv * _cat(agg[...])
    gcn = jnp.dot(aggs, W[...], preferred_element_type=jnp.float32) + b[...]
    out = _leaky(_ln(gcn, g[...], be[...]))
    ident = jnp.dot(xp[...], scW[...], preferred_element_type=jnp.float32) + scb[...]
    xn = out + ident
    xn_ref[...] = xn
    zn_ref[...] = dinv * xn


def _b3_body(agg, degp, xp, W, b, g, be, x4_ref):
    dinv = _dinv_of(degp[...])
    aggs = dinv * _cat(agg[...])
    gcn = jnp.dot(aggs, W[...], preferred_element_type=jnp.float32) + b[...]
    out = _leaky(_ln(gcn, g[...], be[...]))
    x4_ref[...] = out + xp[...]


def _head_body(x4, Wt, cb, rg, rb, hW1, hb1, hW2, hb2, out_ref):
    xb = x4[0]
    zpad = jnp.zeros((2, 256), jnp.float32)
    xpad = jnp.concatenate([zpad, xb, zpad], axis=0)
    y = jnp.zeros((BN, 128), jnp.float32)
    for k in range(5):
        y = y + jnp.dot(lax.slice(xpad, (k, 0), (k + BN, 256)), Wt[...][k],
                        preferred_element_type=jnp.float32)
    y = _leaky(y + cb[...])
    y = _ln(y, rg[...], rb[...])
    mask = jnp.max(jnp.abs(y), axis=-1, keepdims=True) > 1e-8
    y = y * mask.astype(jnp.float32)
    h = _leaky(jnp.dot(y, hW1[...], preferred_element_type=jnp.float32) + hb1[...])
    out_ref[0] = jnp.dot(h, hW2[...], preferred_element_type=jnp.float32) + hb2[...]


def _tc(body, in_specs, out_specs, out_shape, grid=(10,)):
    return pl.pallas_call(body, grid=grid, in_specs=in_specs,
                          out_specs=out_specs, out_shape=out_shape)


def kernel(x, edge_index, batch_size, seq_len,
           gcn_W0, gcn_b0, ln_g0, ln_b0, sc_W0, sc_b0,
           gcn_W1, gcn_b1, ln_g1, ln_b1, sc_W1, sc_b1,
           gcn_W2, gcn_b2, ln_g2, ln_b2, sc_W2, sc_b2,
           gcn_W3, gcn_b3, ln_g3, ln_b3,
           conv_W, conv_b, rln_g, rln_b, h_W1, h_b1, h_W2, h_b2):
    f32 = jnp.float32
    loop = jnp.arange(N, dtype=jnp.int32)
    npad = E_PAD - E_RAW - N
    src_all = jnp.concatenate([edge_index[0], loop,
                               jnp.zeros((npad,), jnp.int32)])
    dst_all = jnp.concatenate([edge_index[1], loop,
                               jnp.full((npad,), DUMMY, jnp.int32)])

    ed2 = jnp.stack([src_all.reshape(-1, EB), dst_all.reshape(-1, EB)], axis=1)

    degp = _make_deg_kernel()(dst_all)

    row1 = lambda n: jnp.reshape(n, (1, -1)).astype(f32)
    b0, g0, be0 = row1(gcn_b0), row1(ln_g0), row1(ln_b0)
    b1, g1, be1 = row1(gcn_b1), row1(ln_g1), row1(ln_b1)
    b2, g2, be2 = row1(gcn_b2), row1(ln_g2), row1(ln_b2)
    b3, g3, be3 = row1(gcn_b3), row1(ln_g3), row1(ln_b3)
    scb0, scb1, scb2 = row1(sc_b0), row1(sc_b1), row1(sc_b2)

    degp_spec = pl.BlockSpec((NC, BN, 16), lambda i: (0, i, 0))

    xw0 = _tc(_mm_body,
              [_row((BN, 128)), _full((128, 64))],
              _row((BN, 64)), jax.ShapeDtypeStruct((N, 64), f32))(x, gcn_W0)
    z0 = _tc(_scale_body,
             [degp_spec, _row((BN, 64))],
             _row((BN, 64)), jax.ShapeDtypeStruct((N, 64), f32))(degp, xw0)

    agg0 = _make_agg_kernel(32, 1)(ed2, z0.reshape(2 * N, 32))

    x1, z1 = _tc(_b0_body,
                 [_agg_spec(32, 1), degp_spec, _row((BN, 128)), _full((128, 64)),
                  _full((1, 64)), _full((1, 64)), _full((1, 64)), _full((1, 64))],
                 [_row((BN, 64)), _row((BN, 64))],
                 [jax.ShapeDtypeStruct((N, 64), f32)] * 2)(
                     agg0, degp, x, sc_W0, scb0, b0, g0, be0)

    agg1 = _make_agg_kernel(32, 1)(ed2, z1.reshape(2 * N, 32))

    x2, z2 = _tc(_bmid_body,
                 [_agg_spec(32, 1), degp_spec, _row((BN, 64)), _full((64, 128)),
                  _full((1, 128)), _full((1, 128)), _full((1, 128)),
                  _full((64, 128)), _full((1, 128))],
                 [_row((BN, 128)), _row((BN, 128))],
                 [jax.ShapeDtypeStruct((N, 128), f32)] * 2)(
                     agg1, degp, x1, gcn_W1, b1, g1, be1, sc_W1, scb1)

    agg2 = _make_agg_kernel(32, 2)(ed2, z2.reshape(4 * N, 32))

    x3, z3 = _tc(_bmid_body,
                 [_agg_spec(32, 2), degp_spec, _row((BN, 128)), _full((128, 256)),
                  _full((1, 256)), _full((1, 256)), _full((1, 256)),
                  _full((128, 256)), _full((1, 256))],
                 [_row((BN, 256)), _row((BN, 256))],
                 [jax.ShapeDtypeStruct((N, 256), f32)] * 2)(
                     agg2, degp, x2, gcn_W2, b2, g2, be2, sc_W2, scb2)

    agg3 = _make_agg_kernel(32, 4, staged=True)(ed2, z3.reshape(8 * N, 32))

    x4 = _tc(_b3_body,
             [_agg_spec(32, 4), degp_spec, _row((BN, 256)), _full((256, 256)),
              _full((1, 256)), _full((1, 256)), _full((1, 256))],
             _row((BN, 256)), jax.ShapeDtypeStruct((N, 256), f32))(
                 agg3, degp, x3, gcn_W3, b3, g3, be3)

    Wt = conv_W.transpose(2, 1, 0)
    coords = _tc(_head_body,
                 [pl.BlockSpec((1, BN, 256), lambda i: (i, 0, 0)),
                  _full((5, 256, 128)), _full((1, 128)), _full((1, 128)),
                  _full((1, 128)), _full((128, 64)), _full((1, 64)),
                  _full((64, 3)), _full((1, 3))],
                 pl.BlockSpec((1, BN, 3), lambda i: (i, 0, 0)),
                 jax.ShapeDtypeStruct((10, BN, 3), f32))(
                     x4.reshape(10, BN, 256), Wt, row1(conv_b), row1(rln_g),
                     row1(rln_b), h_W1, row1(h_b1), h_W2, row1(h_b2))
    return coords
